# baseline (speedup 1.0000x reference)
import math

import jax
import jax.numpy as jnp
from jax.experimental import pallas as pl
from jax.experimental.pallas import tpu as pltpu

_B = 4
_E = 192
_R = 64
_N = _E + _R
_HSZ = 512
_RTOKS = 1000
_PROP = 2
_H = 4
_DH = _HSZ // _H

_BIG = ['Wq', 'Wk', 'Wv', 'Wo', 'W1', 'W2']
_SMALL = ['b1', 'b2', 'a1', 'ln1_g', 'ln1_b', 'ln2_g', 'ln2_b']
_NBIG = _PROP * len(_BIG)


def _dot(a, b):
    return jax.lax.dot_general(
        a, b, (((1,), (0,)), ((), ())), preferred_element_type=jnp.float32
    )


def _dot_nt(a, b):
    return jax.lax.dot_general(
        a, b, (((1,), (1,)), ((), ())), preferred_element_type=jnp.float32
    )


def _layernorm(x, g, b, eps=1e-5):
    m = jnp.mean(x, axis=-1, keepdims=True)
    xc = x - m
    v = jnp.mean(xc * xc, axis=-1, keepdims=True)
    return xc * jax.lax.rsqrt(v + eps) * g + b


def _graph_kernel(rels_ref, vents_ref, adjs_ref, renc_ref, *refs):
    hbm = refs[:_NBIG]
    small = refs[_NBIG:_NBIG + _PROP * len(_SMALL)]
    out_ref = refs[_NBIG + _PROP * len(_SMALL)]
    wbuf = refs[_NBIG + _PROP * len(_SMALL) + 1:-1]
    sems = refs[-1]

    for idx in range(_NBIG):
        pltpu.make_async_copy(hbm[idx], wbuf[idx], sems.at[idx]).start()

    def wg(j, name):
        idx = j * len(_BIG) + _BIG.index(name)
        pltpu.make_async_copy(hbm[idx], wbuf[idx], sems.at[idx]).wait()
        return wbuf[idx][...]

    def sm(j, name):
        return small[j * len(_SMALL) + _SMALL.index(name)][0]

    rels = rels_ref[0]
    ids = jnp.broadcast_to(rels.reshape(_B * _R, 1), (_B * _R, _RTOKS))
    iota = jax.lax.broadcasted_iota(jnp.int32, (_B * _R, _RTOKS), 1)
    onehot = (ids == iota).astype(jnp.float32)
    vrel = _dot(onehot, renc_ref[...])

    pieces = []
    for i in range(_B):
        pieces.append(vents_ref[i])
        pieces.append(vrel[i * _R:(i + 1) * _R])
    vg = jnp.concatenate(pieces, axis=0)

    bias = jnp.where(adjs_ref[...] == 0.0, -1e9, 0.0)
    scale = 1.0 / math.sqrt(_DH)

    for j in range(_PROP):
        q = _dot(vg, wg(j, 'Wq'))
        k = _dot(vg, wg(j, 'Wk'))
        v = _dot(vg, wg(j, 'Wv'))
        outs = []
        for i in range(_B):
            rows = slice(i * _N, (i + 1) * _N)
            bi = bias[i]
            houts = []
            for h in range(_H):
                cols = slice(h * _DH, (h + 1) * _DH)
                s = _dot_nt(q[rows, cols], k[rows, cols]) * scale + bi
                s = s - jnp.max(s, axis=-1, keepdims=True)
                e = jnp.exp(s)
                r = 1.0 / jnp.sum(e, axis=-1, keepdims=True)
                houts.append(_dot(e, v[rows, cols]) * r)
            outs.append(jnp.concatenate(houts, axis=-1))
        o = jnp.concatenate(outs, axis=0)
        attn = _dot(o, wg(j, 'Wo'))
        t = _layernorm(attn, sm(j, 'ln1_g'), sm(j, 'ln1_b'))
        hdn = _dot(t, wg(j, 'W1')) + sm(j, 'b1')
        hdn = jnp.where(hdn >= 0.0, hdn, sm(j, 'a1') * hdn)
        y = _dot(hdn, wg(j, 'W2')) + sm(j, 'b2')
        vg = _layernorm(y + t, sm(j, 'ln2_g'), sm(j, 'ln2_b'))

    out_ref[...] = vg.reshape(_B, _N, _HSZ)


@jax.jit
def _run(adjs, rels, vents, entlens, renc, params):
    rels2 = rels.astype(jnp.int32).reshape(1, _B * _R)
    rep2 = lambda: (0, 0)
    rep3 = lambda: (0, 0, 0)

    big = []
    big_specs = []
    scratch = [pltpu.SemaphoreType.DMA((_NBIG,))]
    wbufs = []
    for j in range(_PROP):
        for f in _BIG:
            w = params[j][f]
            big.append(w)
            big_specs.append(pl.BlockSpec(memory_space=pltpu.MemorySpace.HBM))
            wbufs.append(pltpu.VMEM(w.shape, w.dtype))

    smalls = []
    small_specs = []
    for j in range(_PROP):
        for f in _SMALL:
            w = params[j][f].reshape(1, -1)
            smalls.append(w)
            small_specs.append(pl.BlockSpec(w.shape, rep2))

    in_specs = [
        pl.BlockSpec((1, _B * _R), rep2),
        pl.BlockSpec((_B, _E, _HSZ), rep3),
        pl.BlockSpec((_B, _N, _N), rep3),
        pl.BlockSpec((_RTOKS, _HSZ), rep2),
    ] + big_specs + small_specs

    gents = pl.pallas_call(
        _graph_kernel,
        grid=(),
        in_specs=in_specs,
        out_specs=pl.BlockSpec((_B, _N, _HSZ), rep3),
        out_shape=jax.ShapeDtypeStruct((_B, _N, _HSZ), jnp.float32),
        scratch_shapes=wbufs + scratch,
    )(rels2, vents, adjs, renc, *big, *smalls)

    globv = gents[:, _E, :]
    emask = jnp.arange(_N)[None, :] <= entlens[:, None]
    return (globv, gents, emask)


def kernel(adjs, rels, vents, entlens, renc, params):
    return _run(adjs, rels, vents, entlens, renc, params)
